# baseline (speedup 1.0000x reference)
import functools

import jax
import jax.numpy as jnp
from jax import lax
from jax.experimental import pallas as pl
from jax.experimental.pallas import tpu as pltpu
from jax.experimental.pallas import tpu_sc as plsc

N = 10000
R = 4
E = 80000
IN = 128
OUT = 16
NB = 2

CHUNK = 128
NCHUNK = 80
TILES_PER_REL = 8
EP = TILES_PER_REL * NCHUNK * CHUNK
NP8 = 1280
NPAD = 8 * NP8
DEPTH = 8
GAHEAD = 4
BN = 2000


def _mm_body(xp_ref, bd_ref, coeff_ref, y_ref):
    xb = xp_ref[...]
    for r in range(R):
        wbd = coeff_ref[r, 0] * bd_ref[0] + coeff_ref[r, 1] * bd_ref[1]
        y_ref[r] = jnp.dot(xb, wbd, preferred_element_type=jnp.float32)


MB = NP8 // 5


def _stage1(xp, bases_bd, coeff):
    return pl.pallas_call(
        _mm_body,
        grid=(NP8 // MB,),
        in_specs=[
            pl.BlockSpec((MB, 8 * IN), lambda i: (i, 0)),
            pl.BlockSpec((NB, 8 * IN, 8 * OUT), lambda i: (0, 0, 0)),
            pl.BlockSpec(memory_space=pltpu.SMEM),
        ],
        out_specs=pl.BlockSpec((R, MB, 8 * OUT), lambda i: (0, i, 0)),
        out_shape=jax.ShapeDtypeStruct((R, NP8, 8 * OUT), jnp.float32),
    )(xp, bases_bd, coeff)


_sc_mesh = plsc.VectorSubcoreMesh(core_axis_name="c", subcore_axis_name="s")


@functools.partial(
    pl.kernel,
    out_type=jax.ShapeDtypeStruct((2, NPAD, OUT), jnp.float32),
    mesh=_sc_mesh,
    compiler_params=pltpu.CompilerParams(use_tc_tiling_on_sc=False),
    scratch_types=[
        pltpu.VMEM((NCHUNK, CHUNK), jnp.int32),
        pltpu.VMEM((NCHUNK, CHUNK), jnp.int32),
        pltpu.VMEM_SHARED((NPAD, OUT), jnp.float32),
        pltpu.VMEM_SHARED((2 * NPAD, OUT), jnp.float32),
    ]
    + [pltpu.VMEM((CHUNK, OUT), jnp.float32) for _ in range(DEPTH)]
    + [pltpu.SemaphoreType.DMA for _ in range(2 * DEPTH)],
)
def _sc_scatter(y_hbm, src_hbm, dst_hbm, zeros_hbm, out_hbm,
                src_v, dst_v, acc, y_sp, *rows_and_sems):
    rows_v = rows_and_sems[:DEPTH]
    gsems = rows_and_sems[DEPTH:2 * DEPTH]
    ssems = rows_and_sems[2 * DEPTH:]
    cid = lax.axis_index("c")
    sid = lax.axis_index("s")
    blk = (2 * cid + sid // TILES_PER_REL) * TILES_PER_REL + sid % TILES_PER_REL

    acc16 = acc
    y16 = y_sp

    zrows = NPAD // 16
    pltpu.sync_copy(zeros_hbm.at[pl.ds(sid * zrows, zrows)],
                    acc.at[pl.ds(sid * zrows, zrows)])

    yrows = 2 * NPAD // 16
    pltpu.sync_copy(y_hbm.at[pl.ds(cid * 2 * NPAD + sid * yrows, yrows)],
                    y_sp.at[pl.ds(sid * yrows, yrows)])

    pltpu.sync_copy(src_hbm.at[blk], src_v)
    pltpu.sync_copy(dst_hbm.at[blk], dst_v)
    plsc.subcore_barrier()

    def _wait_gather(b):
        pltpu.make_async_copy(y16.at[pl.ds(0, CHUNK)], rows_v[b],
                              gsems[b]).wait()

    def _wait_scatter(b):
        pltpu.make_async_copy(rows_v[b], acc16.at[dst_v.at[0]],
                              ssems[b]).wait()

    def _gather(g, b):
        pltpu.async_copy(y16.at[src_v.at[g]], rows_v[b], gsems[b])

    def _scatter(c, b):
        pltpu.async_copy(rows_v[b], acc16.at[dst_v.at[c]], ssems[b], add=True)

    for g in range(GAHEAD):
        _gather(g, g)
    for c in range(DEPTH):
        _wait_gather(c)
        _scatter(c, c)
        g = c + GAHEAD
        if g >= DEPTH:
            _wait_scatter(g % DEPTH)
        _gather(g, g % DEPTH)

    def body(i, carry):
        for u in range(DEPTH):
            c = i * DEPTH + u
            _wait_gather(u)
            _scatter(c, u)
            g = c + GAHEAD
            bg = (u + GAHEAD) % DEPTH

            @pl.when(g < NCHUNK)
            def _next():
                _wait_scatter(bg)
                _gather(g, bg)
        return carry

    lax.fori_loop(1, NCHUNK // DEPTH, body, 0)

    for b in range(DEPTH):
        _wait_scatter(b)

    plsc.subcore_barrier()

    pltpu.sync_copy(acc.at[pl.ds(sid * zrows, zrows)],
                    out_hbm.at[cid, pl.ds(sid * zrows, zrows)])


RT = NPAD // 32


@functools.partial(
    pl.kernel,
    out_type=jax.ShapeDtypeStruct((NPAD * OUT,), jnp.float32),
    mesh=_sc_mesh,
    compiler_params=pltpu.CompilerParams(use_tc_tiling_on_sc=False),
    scratch_types=[
        pltpu.VMEM((RT * OUT,), jnp.float32),
        pltpu.VMEM((RT * OUT,), jnp.float32),
        pltpu.VMEM((RT * OUT,), jnp.float32),
    ],
)
def _sc_softmax(part_hbm, out_hbm, p0_v, p1_v, o_v):
    cid = lax.axis_index("c")
    sid = lax.axis_index("s")
    base = (cid * 16 + sid) * RT * OUT
    pltpu.sync_copy(part_hbm.at[0, pl.ds(base, RT * OUT)], p0_v)
    pltpu.sync_copy(part_hbm.at[1, pl.ds(base, RT * OUT)], p1_v)

    iota = lax.iota(jnp.int32, 16)
    perms = [jnp.bitwise_xor(iota, 2 ** b) for b in range(4)]
    _dn = lax.GatherDimensionNumbers(offset_dims=(), collapsed_slice_dims=(0,),
                                     start_index_map=(0,))

    def _shuf(v, p):
        return lax.gather(v, p[:, None], _dn, (1,), unique_indices=True,
                          mode=lax.GatherScatterMode.PROMISE_IN_BOUNDS)

    def _allmax(v):
        for p in perms:
            v = jnp.maximum(v, _shuf(v, p))
        return v

    def _allsum(v):
        for p in perms:
            v = v + _shuf(v, p)
        return v

    def row(i, carry):
        for k in range(4):
            off = (i * 4 + k) * OUT
            v = p0_v[pl.ds(off, OUT)] + p1_v[pl.ds(off, OUT)]
            e = jnp.exp(v - _allmax(v))
            o_v[pl.ds(off, OUT)] = e / _allsum(e)
        return carry

    lax.fori_loop(0, RT // 4, row, 0)
    pltpu.sync_copy(o_v, out_hbm.at[pl.ds(base, RT * OUT)])


SB = NP8


def _sm_body(p_ref, o_ref):
    s = p_ref[0] + p_ref[1]
    parts = []
    for g in range(8):
        sg = s[:, g * OUT:(g + 1) * OUT]
        m = jnp.max(sg, axis=1, keepdims=True)
        e = jnp.exp(sg - m)
        parts.append(e / jnp.sum(e, axis=1, keepdims=True))
    o_ref[...] = jnp.concatenate(parts, axis=1)


def _stage3(partial128):
    return pl.pallas_call(
        _sm_body,
        in_specs=[pl.BlockSpec((2, NP8, 8 * OUT), lambda: (0, 0, 0))],
        out_specs=pl.BlockSpec((NP8, 8 * OUT), lambda: (0, 0)),
        out_shape=jax.ShapeDtypeStruct((NP8, 8 * OUT), jnp.float32),
    )(partial128)


def kernel(x, edge_src, edge_dst, bases, coeff):
    xp = jnp.pad(x, ((0, NPAD - N), (0, 0))).reshape(NP8, 8 * IN)
    eye8 = jnp.eye(8, dtype=jnp.float32)
    bases_bd = (eye8[None, :, None, :, None] *
                bases[:, None, :, None, :]).reshape(NB, 8 * IN, 8 * OUT)
    yp = _stage1(xp, bases_bd, coeff)
    y = yp.reshape(R * NPAD, OUT)

    pad = EP - E
    src = edge_src.astype(jnp.int32)
    dst = edge_dst.astype(jnp.int32)
    src = jnp.pad(src, ((0, 0), (0, pad))) + \
        ((jnp.arange(R, dtype=jnp.int32) % 2) * NPAD)[:, None]
    dst = jnp.pad(dst, ((0, 0), (0, pad)), constant_values=N)
    src_blocks = src.reshape(R * TILES_PER_REL, NCHUNK, CHUNK)
    dst_blocks = dst.reshape(R * TILES_PER_REL, NCHUNK, CHUNK)
    zeros = jnp.zeros((NPAD, OUT), jnp.float32)

    partial = _sc_scatter(y, src_blocks, dst_blocks, zeros)
    sm = _sc_softmax(partial.reshape(2, NPAD * OUT))
    return sm.reshape(NPAD, OUT)[:N]


# speedup vs baseline: 1.0295x; 1.0295x over previous
import functools

import jax
import jax.numpy as jnp
from jax import lax
from jax.experimental import pallas as pl
from jax.experimental.pallas import tpu as pltpu
from jax.experimental.pallas import tpu_sc as plsc

N = 10000
R = 4
E = 80000
IN = 128
OUT = 16
NB = 2

CHUNK = 128
NCHUNK = 80
TILES_PER_REL = 8
EP = TILES_PER_REL * NCHUNK * CHUNK
NP8 = 1280
NPAD = 8 * NP8
DEPTH = 8
GAHEAD = 4
BN = 2000


def _mm_body(xp_ref, bd_ref, coeff_ref, y_ref):
    xb = xp_ref[...]
    for r in range(R):
        wbd = coeff_ref[r, 0] * bd_ref[0] + coeff_ref[r, 1] * bd_ref[1]
        y_ref[r] = jnp.dot(xb, wbd, preferred_element_type=jnp.float32)


MB = NP8 // 2


def _stage1(xp, bases_bd, coeff):
    return pl.pallas_call(
        _mm_body,
        grid=(NP8 // MB,),
        in_specs=[
            pl.BlockSpec((MB, 8 * IN), lambda i: (i, 0)),
            pl.BlockSpec((NB, 8 * IN, 8 * OUT), lambda i: (0, 0, 0)),
            pl.BlockSpec(memory_space=pltpu.SMEM),
        ],
        out_specs=pl.BlockSpec((R, MB, 8 * OUT), lambda i: (0, i, 0)),
        out_shape=jax.ShapeDtypeStruct((R, NP8, 8 * OUT), jnp.float32),
    )(xp, bases_bd, coeff)


_sc_mesh = plsc.VectorSubcoreMesh(core_axis_name="c", subcore_axis_name="s")


@functools.partial(
    pl.kernel,
    out_type=jax.ShapeDtypeStruct((2, NPAD, OUT), jnp.float32),
    mesh=_sc_mesh,
    compiler_params=pltpu.CompilerParams(use_tc_tiling_on_sc=False),
    scratch_types=[
        pltpu.VMEM((NCHUNK, CHUNK), jnp.int32),
        pltpu.VMEM((NCHUNK, CHUNK), jnp.int32),
        pltpu.VMEM_SHARED((NPAD, OUT), jnp.float32),
        pltpu.VMEM_SHARED((2 * NPAD, OUT), jnp.float32),
    ]
    + [pltpu.VMEM((CHUNK, OUT), jnp.float32) for _ in range(DEPTH)]
    + [pltpu.SemaphoreType.DMA for _ in range(2 * DEPTH)],
)
def _sc_scatter(y_hbm, src_hbm, dst_hbm, zeros_hbm, out_hbm,
                src_v, dst_v, acc, y_sp, *rows_and_sems):
    rows_v = rows_and_sems[:DEPTH]
    gsems = rows_and_sems[DEPTH:2 * DEPTH]
    ssems = rows_and_sems[2 * DEPTH:]
    cid = lax.axis_index("c")
    sid = lax.axis_index("s")
    blk = (2 * cid + sid // TILES_PER_REL) * TILES_PER_REL + sid % TILES_PER_REL

    acc16 = acc
    y16 = y_sp

    zrows = NPAD // 16
    pltpu.sync_copy(zeros_hbm.at[pl.ds(sid * zrows, zrows)],
                    acc.at[pl.ds(sid * zrows, zrows)])

    yrows = 2 * NPAD // 16
    pltpu.sync_copy(y_hbm.at[pl.ds(cid * 2 * NPAD + sid * yrows, yrows)],
                    y_sp.at[pl.ds(sid * yrows, yrows)])

    pltpu.sync_copy(src_hbm.at[blk], src_v)
    pltpu.sync_copy(dst_hbm.at[blk], dst_v)
    plsc.subcore_barrier()

    def _wait_gather(b):
        pltpu.make_async_copy(y16.at[pl.ds(0, CHUNK)], rows_v[b],
                              gsems[b]).wait()

    def _wait_scatter(b):
        pltpu.make_async_copy(rows_v[b], acc16.at[dst_v.at[0]],
                              ssems[b]).wait()

    def _gather(g, b):
        pltpu.async_copy(y16.at[src_v.at[g]], rows_v[b], gsems[b])

    def _scatter(c, b):
        pltpu.async_copy(rows_v[b], acc16.at[dst_v.at[c]], ssems[b], add=True)

    for g in range(GAHEAD):
        _gather(g, g)
    for c in range(DEPTH):
        _wait_gather(c)
        _scatter(c, c)
        g = c + GAHEAD
        if g >= DEPTH:
            _wait_scatter(g % DEPTH)
        _gather(g, g % DEPTH)

    def body(i, carry):
        for u in range(DEPTH):
            c = i * DEPTH + u
            _wait_gather(u)
            _scatter(c, u)
            g = c + GAHEAD
            bg = (u + GAHEAD) % DEPTH

            @pl.when(g < NCHUNK)
            def _next():
                _wait_scatter(bg)
                _gather(g, bg)
        return carry

    lax.fori_loop(1, NCHUNK // DEPTH, body, 0)

    for b in range(DEPTH):
        _wait_scatter(b)

    plsc.subcore_barrier()

    pltpu.sync_copy(acc.at[pl.ds(sid * zrows, zrows)],
                    out_hbm.at[cid, pl.ds(sid * zrows, zrows)])


SB = NP8


def _sm_body(p_ref, o_ref):
    s = p_ref[0] + p_ref[1]
    parts = []
    for g in range(8):
        sg = s[:, g * OUT:(g + 1) * OUT]
        m = jnp.max(sg, axis=1, keepdims=True)
        e = jnp.exp(sg - m)
        parts.append(e / jnp.sum(e, axis=1, keepdims=True))
    o_ref[...] = jnp.concatenate(parts, axis=1)


def _stage3(partial128):
    return pl.pallas_call(
        _sm_body,
        in_specs=[pl.BlockSpec((2, NP8, 8 * OUT), lambda: (0, 0, 0))],
        out_specs=pl.BlockSpec((NP8, 8 * OUT), lambda: (0, 0)),
        out_shape=jax.ShapeDtypeStruct((NP8, 8 * OUT), jnp.float32),
    )(partial128)


def kernel(x, edge_src, edge_dst, bases, coeff):
    xp = jnp.pad(x, ((0, NPAD - N), (0, 0))).reshape(NP8, 8 * IN)
    eye8 = jnp.eye(8, dtype=jnp.float32)
    bases_bd = (eye8[None, :, None, :, None] *
                bases[:, None, :, None, :]).reshape(NB, 8 * IN, 8 * OUT)
    yp = _stage1(xp, bases_bd, coeff)
    y = yp.reshape(R * NPAD, OUT)

    pad = EP - E
    src = edge_src.astype(jnp.int32)
    dst = edge_dst.astype(jnp.int32)
    src = jnp.pad(src, ((0, 0), (0, pad))) + \
        ((jnp.arange(R, dtype=jnp.int32) % 2) * NPAD)[:, None]
    dst = jnp.pad(dst, ((0, 0), (0, pad)), constant_values=N)
    src_blocks = src.reshape(R * TILES_PER_REL, NCHUNK, CHUNK)
    dst_blocks = dst.reshape(R * TILES_PER_REL, NCHUNK, CHUNK)
    zeros = jnp.zeros((NPAD, OUT), jnp.float32)

    partial = _sc_scatter(y, src_blocks, dst_blocks, zeros)
    sm = _stage3(partial.reshape(2, NP8, 8 * OUT))
    return sm[:N // 8].reshape(N, OUT)
